# Initial kernel scaffold; baseline (speedup 1.0000x reference)
import jax
import jax.numpy as jnp
from jax.experimental import pallas as pl


def kernel(x, edge_index, batch, W1, b1, W2, b2, W_mf, b_mf, W_bp, b_bp, W_cc, b_cc):
    raise NotImplementedError("write your pallas kernel here")


# speedup vs baseline: 12.0709x; 12.0709x over previous
import functools
import jax
import jax.numpy as jnp
from jax import lax
from jax.experimental import pallas as pl
from jax.experimental.pallas import tpu as pltpu
from jax.experimental.pallas import tpu_sc as plsc

N = 10000
NP = 10240
E = 320000
EP = 327680
D = 128
H = 64
G = 32
NC, NS = 2, 16
NW = NC * NS
EW = EP // NW
CH = 128
NCHUNK = EW // CH
TROWS = NP // NS

@functools.cache
def _sc_mesh():
    return plsc.VectorSubcoreMesh(core_axis_name="c", subcore_axis_name="s",
                                  num_cores=NC, num_subcores=NS)


@functools.cache
def _deg_sc_call():
    return pl.kernel(
        _deg_sc,
        out_type=jax.ShapeDtypeStruct((NC, NP), jnp.float32),
        mesh=_sc_mesh(),
        scratch_types=[
            pltpu.VMEM_SHARED((NP,), jnp.float32),
            pltpu.VMEM((CH,), jnp.int32),
            pltpu.VMEM((CH,), jnp.float32),
        ],
    )


def _deg_sc(dst_hbm, zeros1_hbm, ones_hbm, out_hbm, deg_sh, dstv, onesv):
    c = lax.axis_index("c")
    s = lax.axis_index("s")
    wid = c * NS + s

    pltpu.sync_copy(ones_hbm, onesv)
    pltpu.sync_copy(zeros1_hbm, deg_sh.at[pl.ds(s * TROWS, TROWS)])
    plsc.subcore_barrier()

    def body(j, _):
        base = wid * EW + j * CH
        pltpu.sync_copy(dst_hbm.at[pl.ds(base, CH)], dstv)
        pltpu.sync_copy(onesv, deg_sh.at[dstv], add=True)
        return 0

    lax.fori_loop(0, NCHUNK, body, 0)
    plsc.subcore_barrier()
    pltpu.sync_copy(deg_sh.at[pl.ds(s * TROWS, TROWS)],
                    out_hbm.at[c, pl.ds(s * TROWS, TROWS)])


@functools.cache
def _agg_sc_call():
    return pl.kernel(
        _agg_sc,
        out_type=jax.ShapeDtypeStruct((NC, NP, H), jnp.float32),
        mesh=_sc_mesh(),
        scratch_types=[
            pltpu.VMEM_SHARED((NP, H), jnp.float32),
            pltpu.VMEM((CH,), jnp.int32),
            pltpu.VMEM((CH,), jnp.int32),
            pltpu.VMEM((CH, H), jnp.float32),
            pltpu.SemaphoreType.DMA,
        ],
        compiler_params=pltpu.CompilerParams(use_tc_tiling_on_sc=False),
    )


def _agg_sc(hs_hbm, src_hbm, dst_hbm, zeros2_hbm, out_hbm, acc_sh, srcv,
            dstv, gbuf, sem):
    c = lax.axis_index("c")
    s = lax.axis_index("s")
    wid = c * NS + s

    pltpu.sync_copy(zeros2_hbm, acc_sh.at[pl.ds(s * TROWS, TROWS)])
    plsc.subcore_barrier()

    def body(j, _):
        base = wid * EW + j * CH
        pltpu.sync_copy(src_hbm.at[pl.ds(base, CH)], srcv)
        pltpu.sync_copy(dst_hbm.at[pl.ds(base, CH)], dstv)
        pltpu.async_copy(hs_hbm.at[srcv], gbuf, sem).wait()
        pltpu.sync_copy(gbuf, acc_sh.at[dstv], add=True)
        return 0

    lax.fori_loop(0, NCHUNK, body, 0)
    plsc.subcore_barrier()
    pltpu.sync_copy(acc_sh.at[pl.ds(s * TROWS, TROWS)],
                    out_hbm.at[c, pl.ds(s * TROWS, TROWS)])


_RB = 1024


def _l1_tc(x_ref, w_ref, deg_ref, hs_ref, dinv_ref):
    deg = deg_ref[0, :] + deg_ref[1, :] + 1.0
    dinv = lax.rsqrt(deg)[:, None]
    h = jnp.dot(x_ref[...], w_ref[...], preferred_element_type=jnp.float32)
    hs_ref[...] = h * dinv
    dinv_ref[...] = dinv


def _l2_tc(acc_ref, hs1_ref, dinv_ref, b1_ref, w2_ref, hs2_ref):
    dinv = dinv_ref[...]
    a = acc_ref[0] + acc_ref[1] + hs1_ref[...]
    h = jnp.maximum(a * dinv + b1_ref[...], 0.0)
    h2 = jnp.dot(h, w2_ref[...], preferred_element_type=jnp.float32) * dinv
    rows = lax.broadcasted_iota(jnp.int32, (_RB, H), 0) + pl.program_id(0) * _RB
    hs2_ref[...] = jnp.where(rows < N, h2, 0.0)


def _head_tc(acc_ref, hs2_ref, dinv_ref, b2_ref, batch_ref,
             wmf_ref, bmf_ref, wbp_ref, bbp_ref, wcc_ref, bcc_ref,
             omf_ref, obp_ref, occ_ref):
    a = acc_ref[0] + acc_ref[1] + hs2_ref[...]
    h2 = jnp.maximum(a * dinv_ref[...] + b2_ref[...], 0.0)
    seg = lax.broadcasted_iota(jnp.int32, (G, NP), 0)
    m = (batch_ref[...] == seg).astype(jnp.float32)
    counts = jnp.sum(m, axis=1, keepdims=True)
    pooled = jnp.dot(m, h2, preferred_element_type=jnp.float32)
    pooled = pooled / jnp.maximum(counts, 1.0)
    omf_ref[...] = jnp.dot(pooled, wmf_ref[...],
                           preferred_element_type=jnp.float32) + bmf_ref[...]
    obp_ref[...] = jnp.dot(pooled, wbp_ref[...],
                           preferred_element_type=jnp.float32) + bbp_ref[...]
    occ_ref[...] = jnp.dot(pooled, wcc_ref[...],
                           preferred_element_type=jnp.float32) + bcc_ref[...]


def _layer1(x_p, W1, deg_p):
    grid = NP // _RB
    return pl.pallas_call(
        _l1_tc,
        grid=(grid,),
        in_specs=[
            pl.BlockSpec((_RB, D), lambda i: (i, 0)),
            pl.BlockSpec((D, H), lambda i: (0, 0)),
            pl.BlockSpec((NC, _RB), lambda i: (0, i)),
        ],
        out_specs=[
            pl.BlockSpec((_RB, H), lambda i: (i, 0)),
            pl.BlockSpec((_RB, 1), lambda i: (i, 0)),
        ],
        out_shape=[
            jax.ShapeDtypeStruct((NP, H), jnp.float32),
            jax.ShapeDtypeStruct((NP, 1), jnp.float32),
        ],
    )(x_p, W1, deg_p)


def _layer2(acc1, hs1, dinv, b1, W2):
    grid = NP // _RB
    return pl.pallas_call(
        _l2_tc,
        grid=(grid,),
        in_specs=[
            pl.BlockSpec((NC, _RB, H), lambda i: (0, i, 0)),
            pl.BlockSpec((_RB, H), lambda i: (i, 0)),
            pl.BlockSpec((_RB, 1), lambda i: (i, 0)),
            pl.BlockSpec((1, H), lambda i: (0, 0)),
            pl.BlockSpec((H, H), lambda i: (0, 0)),
        ],
        out_specs=pl.BlockSpec((_RB, H), lambda i: (i, 0)),
        out_shape=jax.ShapeDtypeStruct((NP, H), jnp.float32),
    )(acc1, hs1, dinv, b1, W2)


def _heads(acc2, hs2, dinv, b2, batch_p, W_mf, b_mf, W_bp, b_bp, W_cc, b_cc):
    full = lambda s: pl.BlockSpec(s, lambda: tuple(0 for _ in s))
    return pl.pallas_call(
        _head_tc,
        in_specs=[
            full((NC, NP, H)), full((NP, H)), full((NP, 1)), full((1, H)),
            full((1, NP)),
            full((H, W_mf.shape[1])), full((1, W_mf.shape[1])),
            full((H, W_bp.shape[1])), full((1, W_bp.shape[1])),
            full((H, W_cc.shape[1])), full((1, W_cc.shape[1])),
        ],
        out_specs=[
            full((G, W_mf.shape[1])),
            full((G, W_bp.shape[1])),
            full((G, W_cc.shape[1])),
        ],
        out_shape=[
            jax.ShapeDtypeStruct((G, W_mf.shape[1]), jnp.float32),
            jax.ShapeDtypeStruct((G, W_bp.shape[1]), jnp.float32),
            jax.ShapeDtypeStruct((G, W_cc.shape[1]), jnp.float32),
        ],
    )(acc2, hs2, dinv, b2, batch_p, W_mf, b_mf, W_bp, b_bp, W_cc, b_cc)


@jax.jit
def kernel(x, edge_index, batch, W1, b1, W2, b2, W_mf, b_mf, W_bp, b_bp,
           W_cc, b_cc):
    pad_idx = jnp.full((EP - E,), NP - 1, jnp.int32)
    src_p = jnp.concatenate([edge_index[0], pad_idx])
    dst_p = jnp.concatenate([edge_index[1], pad_idx])
    x_p = jnp.zeros((NP, D), jnp.float32).at[:N].set(x)
    batch_p = jnp.concatenate([batch, jnp.full((NP - N,), G, jnp.int32)])
    batch_p = batch_p[None, :]

    zeros1 = jnp.zeros((TROWS,), jnp.float32)
    zeros2 = jnp.zeros((TROWS, H), jnp.float32)
    ones_c = jnp.ones((CH,), jnp.float32)

    deg_p = _deg_sc_call()(dst_p, zeros1, ones_c)
    hs1, dinv = _layer1(x_p, W1, deg_p)
    acc1 = _agg_sc_call()(hs1, src_p, dst_p, zeros2)
    hs2 = _layer2(acc1, hs1, dinv, b1[None, :], W2)
    acc2 = _agg_sc_call()(hs2, src_p, dst_p, zeros2)
    return _heads(acc2, hs2, dinv, b2[None, :], batch_p,
                  W_mf, b_mf[None, :], W_bp, b_bp[None, :],
                  W_cc, b_cc[None, :])
